# baseline (speedup 1.0000x reference)
import functools

import jax
import jax.numpy as jnp
from jax import lax
from jax.experimental import pallas as pl
from jax.experimental.pallas import tpu as pltpu
from jax.experimental.pallas import tpu_sc as plsc

NUM_WORKERS = 32
CHUNK = 128
GPB = 2
BUFROWS = CHUNK * GPB
NBUF = 3


def kernel(class_ids, table):
    B, T = class_ids.shape
    V, D = table.shape
    total = B * T
    per_w = total // NUM_WORKERS
    n_chunks = per_w // CHUNK
    ids2d = class_ids.reshape(total // CHUNK, CHUNK).astype(jnp.int32)

    mesh = plsc.VectorSubcoreMesh(core_axis_name="c", subcore_axis_name="s")

    n_super = per_w // BUFROWS

    @functools.partial(
        pl.kernel,
        out_type=jax.ShapeDtypeStruct((total, D), jnp.float32),
        mesh=mesh,
        scratch_types=[
            pltpu.VMEM((n_chunks, CHUNK), jnp.int32),
            [pltpu.VMEM((BUFROWS, D), jnp.float32) for _ in range(NBUF)],
            pltpu.VMEM_SHARED((V, D), jnp.float32),
            pltpu.SemaphoreType.DMA((NBUF,)),
            pltpu.SemaphoreType.DMA((NBUF,)),
        ],
    )
    def sc_gather(ids_hbm, table_hbm, out_hbm, idx_v, bufs, table_sh, gsem, ssem):
        wid = lax.axis_index("s") * 2 + lax.axis_index("c")

        @pl.when(lax.axis_index("s") == 0)
        def _():
            pltpu.sync_copy(table_hbm, table_sh)

        plsc.subcore_barrier()

        pltpu.sync_copy(ids_hbm.at[pl.ds(wid * n_chunks, n_chunks)], idx_v)
        base = wid * per_w

        def out_slice(j):
            return out_hbm.at[pl.ds(base + j * BUFROWS, BUFROWS)]

        def gather_start(j, b):
            for u in range(GPB):
                pltpu.make_async_copy(
                    table_sh.at[idx_v.at[j * GPB + u]],
                    bufs[b].at[pl.ds(u * CHUNK, CHUNK)],
                    gsem.at[b]).start()

        def gather_drain(j, b):
            pltpu.make_async_copy(out_slice(j), bufs[b], gsem.at[b]).wait()

        def scatter(j, b):
            return pltpu.make_async_copy(bufs[b], out_slice(j), ssem.at[b])

        for b in range(NBUF):
            gather_start(b, b)
        for j in range(n_super):
            b = j % NBUF
            gather_drain(j, b)
            scatter(j, b).start()
            jp = j - 1
            if jp >= 0:
                scatter(jp, jp % NBUF).wait()
                if jp + NBUF < n_super:
                    gather_start(jp + NBUF, jp % NBUF)
        scatter(n_super - 1, (n_super - 1) % NBUF).wait()

    out = sc_gather(ids2d, table)
    return out.reshape(B, T, D)


# speedup vs baseline: 1.0574x; 1.0574x over previous
import functools

import jax
import jax.numpy as jnp
from jax import lax
from jax.experimental import pallas as pl
from jax.experimental.pallas import tpu as pltpu
from jax.experimental.pallas import tpu_sc as plsc

NUM_WORKERS = 32
CHUNK = 128
NBUF = 6
SKEW = 3


def kernel(class_ids, table):
    B, T = class_ids.shape
    V, D = table.shape
    total = B * T
    per_w = total // NUM_WORKERS
    n_chunks = per_w // CHUNK
    ids2d = class_ids.reshape(total // CHUNK, CHUNK).astype(jnp.int32)

    mesh = plsc.VectorSubcoreMesh(core_axis_name="c", subcore_axis_name="s")

    @functools.partial(
        pl.kernel,
        out_type=jax.ShapeDtypeStruct((total, D), jnp.float32),
        mesh=mesh,
        scratch_types=[
            pltpu.VMEM((n_chunks, CHUNK), jnp.int32),
            [pltpu.VMEM((CHUNK, D), jnp.float32) for _ in range(NBUF)],
            pltpu.VMEM_SHARED((V, D), jnp.float32),
            pltpu.SemaphoreType.DMA((NBUF,)),
            pltpu.SemaphoreType.DMA((NBUF,)),
        ],
    )
    def sc_gather(ids_hbm, table_hbm, out_hbm, idx_v, bufs, table_sh, gsem, ssem):
        wid = lax.axis_index("s") * 2 + lax.axis_index("c")

        @pl.when(lax.axis_index("s") == 0)
        def _():
            pltpu.sync_copy(table_hbm, table_sh)

        plsc.subcore_barrier()

        pltpu.sync_copy(ids_hbm.at[pl.ds(wid * n_chunks, n_chunks)], idx_v)
        base = wid * per_w

        def gather(j, b):
            return pltpu.make_async_copy(
                table_sh.at[idx_v.at[j]], bufs[b], gsem.at[b])

        def scatter(j, b):
            return pltpu.make_async_copy(
                bufs[b], out_hbm.at[pl.ds(base + j * CHUNK, CHUNK)], ssem.at[b])

        for b in range(NBUF):
            gather(b, b).start()
        for j in range(n_chunks):
            b = j % NBUF
            gather(j, b).wait()
            scatter(j, b).start()
            jp = j - SKEW
            if jp >= 0:
                scatter(jp, jp % NBUF).wait()
                if jp + NBUF < n_chunks:
                    gather(jp + NBUF, jp % NBUF).start()
        for j in range(n_chunks - SKEW, n_chunks):
            scatter(j, j % NBUF).wait()

    out = sc_gather(ids2d, table)
    return out.reshape(B, T, D)
